# baseline (speedup 1.0000x reference)
import jax
import jax.numpy as jnp
import numpy as np
from jax import lax
from jax.experimental import pallas as pl
from jax.experimental.pallas import tpu as pltpu
from jax.experimental.pallas import tpu_sc as plsc

_VOCAB = 100000
_D = 1024
_B = 4
_S = 2048
_NTOK = _B * _S

_NC = 2
_NS = 16
_NW = _NC * _NS
_TPW = _NTOK // _NW
_CHUNK = 8
_NBUF = 4
_NCHUNK = _TPW // _CHUNK
_LANES = 16
_NSLICE = _D // _LANES

_SCALE = float(np.sqrt(np.float32(_D)))
_EPS = 1e-5


def _positional_encoding_np(seq_len, d_model):
    pos = np.arange(seq_len, dtype=np.float32)[:, None]
    div = np.exp(
        np.arange(0, d_model, 2, dtype=np.float32)
        * np.float32(-np.log(10000.0) / d_model)
    )
    pe = np.zeros((seq_len, d_model), dtype=np.float32)
    pe[:, 0::2] = np.sin(pos * div)
    pe[:, 1::2] = np.cos(pos * div)
    return pe


_PE_DIV = _positional_encoding_np(_S, _D) * (1.0 / _SCALE)


def _rsqrt(x):
    i = lax.bitcast_convert_type(x, jnp.int32)
    i = jnp.int32(0x5F3759DF) - lax.shift_right_logical(i, 1)
    y = lax.bitcast_convert_type(i, jnp.float32)
    half = jnp.float32(0.5) * x
    for _ in range(3):
        y = y * (jnp.float32(1.5) - half * y * y)
    return y


def _compute_chunk(g_v, p_v, lnw_v, lnb_v):
    def p1(k, accs):
        a, q = accs
        sl = pl.ds(k * _LANES, _LANES)
        na, nq = [], []
        for r in range(_CHUNK):
            e = g_v[r, sl] + p_v[r, sl]
            g_v[r, sl] = e
            na.append(a[r] + e)
            nq.append(q[r] + e * e)
        return (tuple(na), tuple(nq))

    zeros = tuple(jnp.zeros((_LANES,), jnp.float32) for _ in range(_CHUNK))
    a, q = lax.fori_loop(0, _NSLICE, p1, (zeros, zeros))

    rstds, nmus = [], []
    for r in range(_CHUNK):
        s1 = jnp.sum(a[r])
        s2 = jnp.sum(q[r])
        mu = s1 * jnp.float32(_SCALE / _D)
        var = s2 - mu * mu
        rstd = _rsqrt(var + jnp.float32(_EPS))
        rstds.append(rstd * jnp.float32(_SCALE))
        nmus.append(-mu * rstd)

    def p2(k, _):
        sl = pl.ds(k * _LANES, _LANES)
        w = lnw_v[sl]
        bias = lnb_v[sl]
        for r in range(_CHUNK):
            v = g_v[r, sl] * rstds[r] + nmus[r]
            g_v[r, sl] = v * w + bias
        return 0

    lax.fori_loop(0, _NSLICE, p2, 0)


def _sc_body(tok_hbm, pe_hbm, lnw_hbm, lnb_hbm, table_hbm, out_hbm,
             idx_v, g0, g1, g2, g3, pb0, pb1, pb2, pb3, lnw_v, lnb_v,
             ps0, ps1, ps2, ps3, gs0, gs1, gs2, gs3, os0, os1, os2, os3):
    gb = (g0, g1, g2, g3)
    pb = (pb0, pb1, pb2, pb3)
    psm = (ps0, ps1, ps2, ps3)
    gsm = (gs0, gs1, gs2, gs3)
    osm = (os0, os1, os2, os3)

    wid = lax.axis_index("s") * _NC + lax.axis_index("c")
    base = wid * _TPW
    pe_base = lax.rem(base, _S)

    pltpu.sync_copy(tok_hbm.at[pl.ds(base, _TPW)], idx_v)
    pltpu.sync_copy(lnw_hbm, lnw_v)
    pltpu.sync_copy(lnb_hbm, lnb_v)

    def start_pe(c, b):
        pltpu.async_copy(
            pe_hbm.at[pl.ds(pe_base + c * _CHUNK, _CHUNK)], pb[b], psm[b])

    def start_gather(c, b):
        pltpu.async_copy(
            table_hbm.at[idx_v.at[pl.ds(c * _CHUNK, _CHUNK)]], gb[b], gsm[b])

    start_pe(0, 0)
    start_gather(0, 0)
    start_pe(1, 1)
    start_gather(1, 1)

    def iter_body(c4, _):
        for j in range(_NBUF):
            c = c4 * _NBUF + j
            pltpu.make_async_copy(
                table_hbm.at[idx_v.at[pl.ds(c * _CHUNK, _CHUNK)]],
                gb[j], gsm[j]).wait()
            pltpu.make_async_copy(
                pe_hbm.at[pl.ds(0, _CHUNK)], pb[j], psm[j]).wait()

            _compute_chunk(gb[j], pb[j], lnw_v, lnb_v)

            pltpu.async_copy(
                gb[j], out_hbm.at[pl.ds(base + c * _CHUNK, _CHUNK)], osm[j])

            b2 = (j + 2) % _NBUF

            @pl.when(c + 2 < _NCHUNK)
            def _():
                @pl.when(c >= 2)
                def _():
                    pltpu.make_async_copy(
                        gb[b2], out_hbm.at[pl.ds(base, _CHUNK)],
                        osm[b2]).wait()

                start_pe(c + 2, b2)
                start_gather(c + 2, b2)

        return 0

    lax.fori_loop(0, _NCHUNK // _NBUF, iter_body, 0)

    for j in range(_NBUF):
        pltpu.make_async_copy(
            gb[j], out_hbm.at[pl.ds(base, _CHUNK)], osm[j]).wait()


@jax.jit
def _run(tok_flat, table, ln_w, ln_b, pe):
    mesh = plsc.VectorSubcoreMesh(core_axis_name="c", subcore_axis_name="s")
    out = pl.kernel(
        _sc_body,
        out_type=jax.ShapeDtypeStruct((_NTOK, _D), jnp.float32),
        mesh=mesh,
        compiler_params=pltpu.CompilerParams(needs_layout_passes=False),
        scratch_types=[
            pltpu.VMEM((_TPW,), jnp.int32),
            pltpu.VMEM((_CHUNK, _D), jnp.float32),
            pltpu.VMEM((_CHUNK, _D), jnp.float32),
            pltpu.VMEM((_CHUNK, _D), jnp.float32),
            pltpu.VMEM((_CHUNK, _D), jnp.float32),
            pltpu.VMEM((_CHUNK, _D), jnp.float32),
            pltpu.VMEM((_CHUNK, _D), jnp.float32),
            pltpu.VMEM((_CHUNK, _D), jnp.float32),
            pltpu.VMEM((_CHUNK, _D), jnp.float32),
            pltpu.VMEM((_D,), jnp.float32),
            pltpu.VMEM((_D,), jnp.float32),
        ] + [pltpu.SemaphoreType.DMA] * 12,
    )(tok_flat, pe, ln_w, ln_b, table)
    return out


def kernel(token_ids, table, ln_w, ln_b):
    pe = jnp.asarray(_PE_DIV)
    tok_flat = token_ids.reshape(_NTOK).astype(jnp.int32)
    out = _run(tok_flat, table, ln_w, ln_b, pe)
    return out.reshape(_B, _S, _D)


# speedup vs baseline: 1.0504x; 1.0504x over previous
import jax
import jax.numpy as jnp
import numpy as np
from jax import lax
from jax.experimental import pallas as pl
from jax.experimental.pallas import tpu as pltpu
from jax.experimental.pallas import tpu_sc as plsc

_VOCAB = 100000
_D = 1024
_B = 4
_S = 2048
_NTOK = _B * _S

_NC = 2
_NS = 16
_NW = _NC * _NS
_TPW = _NTOK // _NW
_CHUNK = 8
_NBUF = 4
_NCHUNK = _TPW // _CHUNK
_LANES = 16
_NSLICE = _D // _LANES

_SCALE = float(np.sqrt(np.float32(_D)))
_EPS = 1e-5


def _positional_encoding_np(seq_len, d_model):
    pos = np.arange(seq_len, dtype=np.float32)[:, None]
    div = np.exp(
        np.arange(0, d_model, 2, dtype=np.float32)
        * np.float32(-np.log(10000.0) / d_model)
    )
    pe = np.zeros((seq_len, d_model), dtype=np.float32)
    pe[:, 0::2] = np.sin(pos * div)
    pe[:, 1::2] = np.cos(pos * div)
    return pe


_PE_DIV = _positional_encoding_np(_S, _D) * (1.0 / _SCALE)


def _rsqrt(x):
    i = lax.bitcast_convert_type(x, jnp.int32)
    i = jnp.int32(0x5F3759DF) - lax.shift_right_logical(i, 1)
    y = lax.bitcast_convert_type(i, jnp.float32)
    half = jnp.float32(0.5) * x
    for _ in range(3):
        y = y * (jnp.float32(1.5) - half * y * y)
    return y


def _compute_chunk(g_v, p_v, lnw_v, lnb_v):
    def p1(k, accs):
        a, q = accs
        sl = pl.ds(k * _LANES, _LANES)
        na, nq = [], []
        for r in range(_CHUNK):
            e = g_v[r, sl] + p_v[r, sl]
            na.append(a[r] + e)
            nq.append(q[r] + e * e)
        return (tuple(na), tuple(nq))

    zeros = tuple(jnp.zeros((_LANES,), jnp.float32) for _ in range(_CHUNK))
    a, q = lax.fori_loop(0, _NSLICE, p1, (zeros, zeros))

    rstds, nmus = [], []
    for r in range(_CHUNK):
        s1 = jnp.sum(a[r])
        s2 = jnp.sum(q[r])
        mu = s1 * jnp.float32(_SCALE / _D)
        var = s2 - mu * mu
        rstd = _rsqrt(var + jnp.float32(_EPS))
        rstds.append(rstd * jnp.float32(_SCALE))
        nmus.append(-mu * rstd)

    def p2(k, _):
        sl = pl.ds(k * _LANES, _LANES)
        w = lnw_v[sl]
        bias = lnb_v[sl]
        for r in range(_CHUNK):
            e = g_v[r, sl] + p_v[r, sl]
            v = e * rstds[r] + nmus[r]
            g_v[r, sl] = v * w + bias
        return 0

    lax.fori_loop(0, _NSLICE, p2, 0)


def _sc_body(tok_hbm, pe_hbm, lnw_hbm, lnb_hbm, table_hbm, out_hbm,
             idx_v, g0, g1, g2, g3, pb0, pb1, pb2, pb3, lnw_v, lnb_v,
             ps0, ps1, ps2, ps3, gs0, gs1, gs2, gs3, os0, os1, os2, os3):
    gb = (g0, g1, g2, g3)
    pb = (pb0, pb1, pb2, pb3)
    psm = (ps0, ps1, ps2, ps3)
    gsm = (gs0, gs1, gs2, gs3)
    osm = (os0, os1, os2, os3)

    wid = lax.axis_index("s") * _NC + lax.axis_index("c")
    base = wid * _TPW
    pe_base = lax.rem(base, _S)

    pltpu.sync_copy(tok_hbm.at[pl.ds(base, _TPW)], idx_v)
    pltpu.sync_copy(lnw_hbm, lnw_v)
    pltpu.sync_copy(lnb_hbm, lnb_v)

    def start_pe(c, b):
        pltpu.async_copy(
            pe_hbm.at[pl.ds(pe_base + c * _CHUNK, _CHUNK)], pb[b], psm[b])

    def start_gather(c, b):
        pltpu.async_copy(
            table_hbm.at[idx_v.at[pl.ds(c * _CHUNK, _CHUNK)]], gb[b], gsm[b])

    start_pe(0, 0)
    start_gather(0, 0)
    start_pe(1, 1)
    start_gather(1, 1)

    def iter_body(c4, _):
        for j in range(_NBUF):
            c = c4 * _NBUF + j
            pltpu.make_async_copy(
                table_hbm.at[idx_v.at[pl.ds(c * _CHUNK, _CHUNK)]],
                gb[j], gsm[j]).wait()
            pltpu.make_async_copy(
                pe_hbm.at[pl.ds(0, _CHUNK)], pb[j], psm[j]).wait()

            _compute_chunk(gb[j], pb[j], lnw_v, lnb_v)

            pltpu.async_copy(
                gb[j], out_hbm.at[pl.ds(base + c * _CHUNK, _CHUNK)], osm[j])

            b2 = (j + 2) % _NBUF

            @pl.when(c + 2 < _NCHUNK)
            def _():
                @pl.when(c >= 2)
                def _():
                    pltpu.make_async_copy(
                        gb[b2], out_hbm.at[pl.ds(base, _CHUNK)],
                        osm[b2]).wait()

                start_pe(c + 2, b2)
                start_gather(c + 2, b2)

        return 0

    lax.fori_loop(0, _NCHUNK // _NBUF, iter_body, 0)

    for j in range(_NBUF):
        pltpu.make_async_copy(
            gb[j], out_hbm.at[pl.ds(base, _CHUNK)], osm[j]).wait()


@jax.jit
def _run(tok_flat, table, ln_w, ln_b, pe):
    mesh = plsc.VectorSubcoreMesh(core_axis_name="c", subcore_axis_name="s")
    out = pl.kernel(
        _sc_body,
        out_type=jax.ShapeDtypeStruct((_NTOK, _D), jnp.float32),
        mesh=mesh,
        compiler_params=pltpu.CompilerParams(needs_layout_passes=False),
        scratch_types=[
            pltpu.VMEM((_TPW,), jnp.int32),
            pltpu.VMEM((_CHUNK, _D), jnp.float32),
            pltpu.VMEM((_CHUNK, _D), jnp.float32),
            pltpu.VMEM((_CHUNK, _D), jnp.float32),
            pltpu.VMEM((_CHUNK, _D), jnp.float32),
            pltpu.VMEM((_CHUNK, _D), jnp.float32),
            pltpu.VMEM((_CHUNK, _D), jnp.float32),
            pltpu.VMEM((_CHUNK, _D), jnp.float32),
            pltpu.VMEM((_CHUNK, _D), jnp.float32),
            pltpu.VMEM((_D,), jnp.float32),
            pltpu.VMEM((_D,), jnp.float32),
        ] + [pltpu.SemaphoreType.DMA] * 12,
    )(tok_flat, pe, ln_w, ln_b, table)
    return out


def kernel(token_ids, table, ln_w, ln_b):
    pe = jnp.asarray(_PE_DIV)
    tok_flat = token_ids.reshape(_NTOK).astype(jnp.int32)
    out = _run(tok_flat, table, ln_w, ln_b, pe)
    return out.reshape(_B, _S, _D)
